# baseline (speedup 1.0000x reference)
import functools

import jax
import jax.numpy as jnp
from jax import lax
from jax.experimental import pallas as pl
from jax.experimental.pallas import tpu as pltpu
from jax.experimental.pallas import tpu_sc as plsc

N_ATOMS = 1600000
N_SEG = 10000
OUT_U = 16
L = 16

N_CORES = 2
N_SUBCORES = 16
NW = N_CORES * N_SUBCORES
T_TILES = N_ATOMS // 128
TB = 10
BLK_ATOMS = TB * 128
NBLOCKS = T_TILES // TB
FULL_BLK = NBLOCKS // NW
EXTRA_W = NBLOCKS - FULL_BLK * NW

CHUNKS_PER_TILE = 128 // L
STG = 1408
DUMMY = 10100
SEG_PER_TILE = 632
N_SEG_PAD = SEG_PER_TILE * N_SUBCORES

IDS_OFF = 8
IDS_LEN = IDS_OFF + BLK_ATOMS + 24

_mesh = plsc.VectorSubcoreMesh(core_axis_name="c", subcore_axis_name="s")

_GDN = lax.GatherDimensionNumbers(
    offset_dims=(), collapsed_slice_dims=(0,), start_index_map=(0,)
)


def _gather16(v, idx):
    return lax.gather(
        v, idx[:, None], _GDN, (1,),
        mode=lax.GatherScatterMode.PROMISE_IN_BOUNDS,
    )


@functools.partial(
    pl.kernel,
    out_type=jax.ShapeDtypeStruct((N_CORES, N_SEG_PAD, OUT_U), jnp.float32),
    mesh=_mesh,
    scratch_types=[
        pltpu.VMEM_SHARED((N_SEG_PAD, OUT_U), jnp.float32),
        pltpu.VMEM((N_CORES, TB, 8, 128), jnp.float32),
        pltpu.VMEM((IDS_LEN,), jnp.int32),
        pltpu.VMEM((STG, OUT_U), jnp.float32),
        pltpu.VMEM((STG,), jnp.int32),
    ],
    compiler_params=pltpu.CompilerParams(use_tc_tiling_on_sc=False, needs_layout_passes=False),
)
def _sc_segsum(ids_hbm, x_hbm, out_hbm, acc_sh, xv, ids_v, stg_v, idt_v):
    c = lax.axis_index("c")
    s = lax.axis_index("s")
    w = c * N_SUBCORES + s

    iota = lax.iota(jnp.int32, L)
    zrow = jnp.zeros((OUT_U,), jnp.float32)
    dummy_vec = jnp.full((L,), DUMMY, jnp.int32)
    fifteen = jnp.full((L,), 15, jnp.int32)
    shift_idx = [jnp.maximum(iota - d, 0) for d in (1, 2, 4, 8)]
    ge_d = [iota >= d for d in (1, 2, 4, 8)]
    fzero = jnp.zeros((L,), jnp.float32)

    ids_v[pl.ds(0, L)] = jnp.full((L,), -1, jnp.int32)
    ids_v[pl.ds(IDS_OFF + BLK_ATOMS, L)] = jnp.full((L,), -2, jnp.int32)

    def _zero_stg(i, carry):
        stg_v[i, :] = zrow
        return carry

    lax.fori_loop(0, STG, _zero_stg, 0)

    def _dummy_idt(j, carry):
        idt_v[pl.ds(j * L, L)] = dummy_vec
        return carry

    lax.fori_loop(0, STG // L, _dummy_idt, 0)

    pltpu.sync_copy(stg_v.at[pl.ds(0, SEG_PER_TILE)],
                    acc_sh.at[pl.ds(s * SEG_PER_TILE, SEG_PER_TILE)])
    plsc.subcore_barrier()

    def _chunk(t_local, c_i, n_vec):
        pos = IDS_OFF + t_local * 128 + c_i * L
        ids16 = ids_v[pl.ds(pos, L)]
        prev16 = ids_v[pl.ds(pos - 1, L)]
        next16 = ids_v[pl.ds(pos + 1, L)]
        chg_b = ids16 != prev16
        cum_chg = plsc.cumsum(jnp.where(chg_b, 1, 0))
        rank16 = n_vec + cum_chg
        same = [
            ge_d[k] & (ids16 == _gather16(ids16, shift_idx[k]))
            for k in range(4)
        ]
        wmask = (ids16 != next16) | (iota == 15)
        for g in range(N_CORES):
            for r in range(8):
                u = 8 * g + r
                val = xv[g, t_local, r, pl.ds(c_i * L, L)]
                cs = val
                for k in range(4):
                    cs = cs + jnp.where(same[k], _gather16(cs, shift_idx[k]), fzero)
                plsc.addupdate_scatter(
                    stg_v,
                    [rank16, jnp.full((L,), u, jnp.int32)],
                    cs,
                    mask=wmask,
                )
        plsc.store_scatter(idt_v, [rank16], ids16, mask=wmask)
        return n_vec + _gather16(cum_chg, fifteen)

    def _block(i, carry):
        b = w + NW * i
        pltpu.sync_copy(ids_hbm.at[pl.ds(b * BLK_ATOMS, BLK_ATOMS)],
                        ids_v.at[pl.ds(IDS_OFF, BLK_ATOMS)])
        pltpu.sync_copy(x_hbm.at[:, pl.ds(b * TB, TB)], xv)

        def _tile(t_local, n_vec):
            for c_i in range(CHUNKS_PER_TILE):
                n_vec = _chunk(t_local, c_i, n_vec)
            return n_vec

        n_vec = lax.fori_loop(0, TB, _tile, jnp.zeros((L,), jnp.int32))
        n_used = jnp.max(n_vec) + 1
        n_fc = (n_used + 127) // 128

        def _flush(j, carry2):
            pltpu.sync_copy(
                stg_v.at[pl.ds(j * 128, 128)],
                acc_sh.at[idt_v.at[pl.ds(j * 128, 128)]],
                add=True,
            )
            return carry2

        lax.fori_loop(0, n_fc, _flush, 0)

        def _rezero(rr, carry2):
            stg_v[rr, :] = zrow
            return carry2

        lax.fori_loop(0, n_fc * 128, _rezero, 0)

        def _redummy(j, carry2):
            idt_v[pl.ds(j * L, L)] = dummy_vec
            return carry2

        lax.fori_loop(0, n_fc * (128 // L), _redummy, 0)
        return carry

    nblk = jnp.where(w < EXTRA_W, FULL_BLK + 1, FULL_BLK)
    lax.fori_loop(0, nblk, _block, 0)
    plsc.subcore_barrier()

    pltpu.sync_copy(
        acc_sh.at[pl.ds(s * SEG_PER_TILE, SEG_PER_TILE)],
        out_hbm.at[c, pl.ds(s * SEG_PER_TILE, SEG_PER_TILE)],
    )


CMB_ROWS = 320
CMB_TAIL = N_SEG - (NW - 1) * CMB_ROWS


@functools.partial(
    pl.kernel,
    out_type=jax.ShapeDtypeStruct((N_SEG, OUT_U), jnp.float32),
    mesh=_mesh,
    scratch_types=[
        pltpu.VMEM((CMB_ROWS, OUT_U), jnp.float32),
        pltpu.VMEM((CMB_ROWS, OUT_U), jnp.float32),
    ],
    compiler_params=pltpu.CompilerParams(use_tc_tiling_on_sc=False, needs_layout_passes=False),
)
def _sc_combine(p_hbm, out_hbm, a_v, b_v):
    c = lax.axis_index("c")
    s = lax.axis_index("s")
    w = c * N_SUBCORES + s
    base = w * CMB_ROWS

    def _do(nrows):
        pltpu.sync_copy(p_hbm.at[0, pl.ds(base, nrows)], a_v.at[pl.ds(0, nrows)])
        pltpu.sync_copy(p_hbm.at[1, pl.ds(base, nrows)], b_v.at[pl.ds(0, nrows)])

        def _add(i, carry):
            a_v[i, :] = a_v[i, :] + b_v[i, :]
            return carry

        lax.fori_loop(0, nrows, _add, 0)
        pltpu.sync_copy(a_v.at[pl.ds(0, nrows)], out_hbm.at[pl.ds(base, nrows)])

    @pl.when(w < NW - 1)
    def _full():
        _do(CMB_ROWS)

    @pl.when(w == NW - 1)
    def _tail():
        _do(CMB_TAIL)


def kernel(ind_1, output):
    batch = ind_1[:, 0] if ind_1.ndim == 2 else ind_1
    ids = batch.astype(jnp.int32)
    x = output.reshape(T_TILES, 128, N_CORES, 8).transpose(2, 0, 3, 1)
    partials = _sc_segsum(ids, x)
    return _sc_combine(partials)


# speedup vs baseline: 3.3320x; 3.3320x over previous
import functools

import jax
import jax.numpy as jnp
from jax import lax
from jax.experimental import pallas as pl
from jax.experimental.pallas import tpu as pltpu
from jax.experimental.pallas import tpu_sc as plsc

N_ATOMS = 1600000
N_SEG = 10000
OUT_U = 16
L = 16

N_CORES = 2
N_SUBCORES = 16
NW = N_CORES * N_SUBCORES
T_TILES = N_ATOMS // 128
TB = 10
BLK_ATOMS = TB * 128
NBLOCKS = T_TILES // TB
FULL_BLK = NBLOCKS // NW
EXTRA_W = NBLOCKS - FULL_BLK * NW

CHUNKS_PER_TILE = 128 // L
STG = 1408
DUMMY = 10100
SEG_PER_TILE = 632
N_SEG_PAD = SEG_PER_TILE * N_SUBCORES

IDS_OFF = 8
IDS_LEN = IDS_OFF + BLK_ATOMS + 24

_mesh = plsc.VectorSubcoreMesh(core_axis_name="c", subcore_axis_name="s")

_GDN = lax.GatherDimensionNumbers(
    offset_dims=(), collapsed_slice_dims=(0,), start_index_map=(0,)
)


def _gather16(v, idx):
    return lax.gather(
        v, idx[:, None], _GDN, (1,),
        mode=lax.GatherScatterMode.PROMISE_IN_BOUNDS,
    )


@functools.partial(
    pl.kernel,
    out_type=jax.ShapeDtypeStruct((N_CORES, N_SEG_PAD, OUT_U), jnp.float32),
    mesh=_mesh,
    scratch_types=[
        pltpu.VMEM_SHARED((N_SEG_PAD, OUT_U), jnp.float32),
        pltpu.VMEM((N_CORES, TB, 8, 128), jnp.float32),
        pltpu.VMEM((IDS_LEN,), jnp.int32),
        pltpu.VMEM((STG, OUT_U), jnp.float32),
        pltpu.VMEM((STG,), jnp.int32),
    ],
    compiler_params=pltpu.CompilerParams(use_tc_tiling_on_sc=False, needs_layout_passes=False),
)
def _sc_segsum(ids_hbm, x_hbm, out_hbm, acc_sh, xv, ids_v, stg_v, idt_v):
    c = lax.axis_index("c")
    s = lax.axis_index("s")
    w = c * N_SUBCORES + s

    iota = lax.iota(jnp.int32, L)
    zrow = jnp.zeros((OUT_U,), jnp.float32)
    dummy_vec = jnp.full((L,), DUMMY, jnp.int32)
    fifteen = jnp.full((L,), 15, jnp.int32)
    zero16 = jnp.zeros((L,), jnp.int32)
    fzero = jnp.zeros((L,), jnp.float32)
    ucols = [jnp.full((L,), u, jnp.int32) for u in range(OUT_U)]

    ids_v[pl.ds(0, L)] = jnp.full((L,), -1, jnp.int32)
    ids_v[pl.ds(IDS_OFF + BLK_ATOMS, L)] = jnp.full((L,), -2, jnp.int32)

    def _zero_stg(i, carry):
        stg_v[i, :] = zrow
        return carry

    lax.fori_loop(0, STG, _zero_stg, 0)

    def _dummy_idt(j, carry):
        idt_v[pl.ds(j * L, L)] = dummy_vec
        return carry

    lax.fori_loop(0, STG // L, _dummy_idt, 0)

    pltpu.sync_copy(stg_v.at[pl.ds(0, SEG_PER_TILE)],
                    acc_sh.at[pl.ds(s * SEG_PER_TILE, SEG_PER_TILE)])
    plsc.subcore_barrier()

    def _chunk_rank(t_local, c_i, n_vec):
        pos = IDS_OFF + t_local * 128 + c_i * L
        ids16 = ids_v[pl.ds(pos, L)]
        prev16 = ids_v[pl.ds(pos - 1, L)]
        chg_b = ids16 != prev16
        cum_chg = plsc.cumsum(jnp.where(chg_b, 1, 0))
        rank16 = n_vec + cum_chg
        for g in range(N_CORES):
            for r in range(8):
                val = xv[g, t_local, r, pl.ds(c_i * L, L)]
                plsc.addupdate_scatter(stg_v, [rank16, ucols[8 * g + r]], val)
        plsc.store_scatter(idt_v, [rank16], ids16)
        return n_vec + _gather16(cum_chg, fifteen)

    def _chunk_fast(t_local, c_i, id0_vec, car):
        pos = IDS_OFF + t_local * 128 + c_i * L
        ids16 = ids_v[pl.ds(pos, L)]
        prev16 = ids_v[pl.ds(pos - 1, L)]
        uniform = jnp.all(ids16 == prev16)

        def _fast(car2):
            open_row = car2[0]
            accs = list(car2[1:])
            for g in range(N_CORES):
                for r in range(8):
                    u = 8 * g + r
                    accs[u] = accs[u] + xv[g, t_local, r, pl.ds(c_i * L, L)]
            return (open_row, *accs)

        def _slow(car2):
            open_row = car2[0]
            accs = car2[1:]
            last_vec = _gather16(ids16, fifteen)
            trail_b = ids16 == last_vec
            keep_b = jnp.logical_not(trail_b)
            row16 = ids16 - id0_vec
            new_accs = []
            for g in range(N_CORES):
                for r in range(8):
                    u = 8 * g + r
                    val = xv[g, t_local, r, pl.ds(c_i * L, L)]
                    plsc.addupdate_scatter(stg_v, [open_row, ucols[u]], accs[u])
                    plsc.addupdate_scatter(
                        stg_v, [row16, ucols[u]], val, mask=keep_b
                    )
                    new_accs.append(jnp.where(trail_b, val, fzero))
            return (last_vec - id0_vec, *new_accs)

        return lax.cond(uniform, _fast, _slow, car)

    def _block(i, carry):
        b = w + NW * i
        pltpu.sync_copy(ids_hbm.at[pl.ds(b * BLK_ATOMS, BLK_ATOMS)],
                        ids_v.at[pl.ds(IDS_OFF, BLK_ATOMS)])
        pltpu.sync_copy(x_hbm.at[:, pl.ds(b * TB, TB)], xv)

        first16 = ids_v[pl.ds(IDS_OFF, L)]
        id0_vec = _gather16(first16, zero16)
        last16 = ids_v[pl.ds(IDS_OFF + BLK_ATOMS - L, L)]
        idlast_vec = _gather16(last16, fifteen)
        blk_range = jnp.max(idlast_vec - id0_vec) + 1

        def _idt(j, carry2):
            idt_v[pl.ds(j * L, L)] = jnp.minimum(
                id0_vec + (iota + j * L), dummy_vec
            )
            return carry2

        lax.fori_loop(0, STG // L, _idt, 0)

        def _normal(_):
            def _tile(t_local, car):
                for c_i in range(CHUNKS_PER_TILE):
                    car = _chunk_fast(t_local, c_i, id0_vec, car)
                return car

            init = (jnp.full((L,), STG - 1, jnp.int32),) + tuple(
                fzero for _ in range(OUT_U)
            )
            car = lax.fori_loop(0, TB, _tile, init)
            open_row = car[0]
            accs = car[1:]
            for u in range(OUT_U):
                plsc.addupdate_scatter(stg_v, [open_row, ucols[u]], accs[u])
            return blk_range

        def _fallback(_):
            def _tile(t_local, n_vec):
                for c_i in range(CHUNKS_PER_TILE):
                    n_vec = _chunk_rank(t_local, c_i, n_vec)
                return n_vec

            n_vec = lax.fori_loop(0, TB, _tile, zero16)
            return jnp.max(n_vec) + 1

        n_rows = lax.cond(blk_range > STG - 8, _fallback, _normal, 0)
        n_fc = (n_rows + 127) // 128

        def _flush(j, carry2):
            pltpu.sync_copy(
                stg_v.at[pl.ds(j * 128, 128)],
                acc_sh.at[idt_v.at[pl.ds(j * 128, 128)]],
                add=True,
            )
            return carry2

        lax.fori_loop(0, n_fc, _flush, 0)

        def _rezero(rr, carry2):
            stg_v[rr, :] = zrow
            return carry2

        lax.fori_loop(0, n_fc * 128, _rezero, 0)
        return carry

    nblk = jnp.where(w < EXTRA_W, FULL_BLK + 1, FULL_BLK)
    lax.fori_loop(0, nblk, _block, 0)
    plsc.subcore_barrier()

    pltpu.sync_copy(
        acc_sh.at[pl.ds(s * SEG_PER_TILE, SEG_PER_TILE)],
        out_hbm.at[c, pl.ds(s * SEG_PER_TILE, SEG_PER_TILE)],
    )


CMB_ROWS = 320
CMB_TAIL = N_SEG - (NW - 1) * CMB_ROWS


@functools.partial(
    pl.kernel,
    out_type=jax.ShapeDtypeStruct((N_SEG, OUT_U), jnp.float32),
    mesh=_mesh,
    scratch_types=[
        pltpu.VMEM((CMB_ROWS, OUT_U), jnp.float32),
        pltpu.VMEM((CMB_ROWS, OUT_U), jnp.float32),
    ],
    compiler_params=pltpu.CompilerParams(use_tc_tiling_on_sc=False, needs_layout_passes=False),
)
def _sc_combine(p_hbm, out_hbm, a_v, b_v):
    c = lax.axis_index("c")
    s = lax.axis_index("s")
    w = c * N_SUBCORES + s
    base = w * CMB_ROWS

    def _do(nrows):
        pltpu.sync_copy(p_hbm.at[0, pl.ds(base, nrows)], a_v.at[pl.ds(0, nrows)])
        pltpu.sync_copy(p_hbm.at[1, pl.ds(base, nrows)], b_v.at[pl.ds(0, nrows)])

        def _add(i, carry):
            a_v[i, :] = a_v[i, :] + b_v[i, :]
            return carry

        lax.fori_loop(0, nrows, _add, 0)
        pltpu.sync_copy(a_v.at[pl.ds(0, nrows)], out_hbm.at[pl.ds(base, nrows)])

    @pl.when(w < NW - 1)
    def _full():
        _do(CMB_ROWS)

    @pl.when(w == NW - 1)
    def _tail():
        _do(CMB_TAIL)


def kernel(ind_1, output):
    batch = ind_1[:, 0] if ind_1.ndim == 2 else ind_1
    ids = batch.astype(jnp.int32)
    x = output.reshape(T_TILES, 128, N_CORES, 8).transpose(2, 0, 3, 1)
    partials = _sc_segsum(ids, x)
    return _sc_combine(partials)


# speedup vs baseline: 4.0310x; 1.2098x over previous
import functools

import jax
import jax.numpy as jnp
from jax import lax
from jax.experimental import pallas as pl
from jax.experimental.pallas import tpu as pltpu
from jax.experimental.pallas import tpu_sc as plsc

N_ATOMS = 1600000
N_SEG = 10000
OUT_U = 16
L = 16

N_CORES = 2
N_SUBCORES = 16
NW = N_CORES * N_SUBCORES
T_TILES = N_ATOMS // 128
TB = 10
BLK_ATOMS = TB * 128
NBLOCKS = T_TILES // TB
FULL_BLK = NBLOCKS // NW
EXTRA_W = NBLOCKS - FULL_BLK * NW

CHUNKS_PER_TILE = 128 // L
STG = 1408
DUMMY = 10100
SEG_PER_TILE = 632
N_SEG_PAD = SEG_PER_TILE * N_SUBCORES

IDS_OFF = 8
IDS_LEN = IDS_OFF + BLK_ATOMS + 24

_mesh = plsc.VectorSubcoreMesh(core_axis_name="c", subcore_axis_name="s")

_GDN = lax.GatherDimensionNumbers(
    offset_dims=(), collapsed_slice_dims=(0,), start_index_map=(0,)
)


def _gather16(v, idx):
    return lax.gather(
        v, idx[:, None], _GDN, (1,),
        mode=lax.GatherScatterMode.PROMISE_IN_BOUNDS,
    )


@functools.partial(
    pl.kernel,
    out_type=jax.ShapeDtypeStruct((N_CORES, N_SEG_PAD, OUT_U), jnp.float32),
    mesh=_mesh,
    scratch_types=[
        pltpu.VMEM_SHARED((N_SEG_PAD, OUT_U), jnp.float32),
        pltpu.VMEM((2, N_CORES, TB, 8, 128), jnp.float32),
        pltpu.VMEM((2, IDS_LEN), jnp.int32),
        pltpu.VMEM((STG, OUT_U), jnp.float32),
        pltpu.VMEM((STG,), jnp.int32),
        pltpu.SemaphoreType.DMA((2,)),
    ],
    compiler_params=pltpu.CompilerParams(use_tc_tiling_on_sc=False, needs_layout_passes=False),
)
def _sc_segsum(ids_hbm, x_hbm, out_hbm, acc_sh, xv, ids_v, stg_v, idt_v, sem):
    c = lax.axis_index("c")
    s = lax.axis_index("s")
    w = c * N_SUBCORES + s

    iota = lax.iota(jnp.int32, L)
    zrow = jnp.zeros((OUT_U,), jnp.float32)
    dummy_vec = jnp.full((L,), DUMMY, jnp.int32)
    fifteen = jnp.full((L,), 15, jnp.int32)
    zero16 = jnp.zeros((L,), jnp.int32)
    fzero = jnp.zeros((L,), jnp.float32)
    ucols = [jnp.full((L,), u, jnp.int32) for u in range(OUT_U)]

    ids_v[0, pl.ds(0, L)] = jnp.full((L,), -1, jnp.int32)
    ids_v[1, pl.ds(0, L)] = jnp.full((L,), -1, jnp.int32)

    def _zero_stg(i, carry):
        stg_v[i, :] = zrow
        return carry

    lax.fori_loop(0, STG, _zero_stg, 0)

    def _dummy_idt(j, carry):
        idt_v[pl.ds(j * L, L)] = dummy_vec
        return carry

    lax.fori_loop(0, STG // L, _dummy_idt, 0)

    pltpu.sync_copy(stg_v.at[pl.ds(0, SEG_PER_TILE)],
                    acc_sh.at[pl.ds(s * SEG_PER_TILE, SEG_PER_TILE)])
    plsc.subcore_barrier()

    def _chunk_rank(buf, t_local, c_i, n_vec):
        pos = IDS_OFF + t_local * 128 + c_i * L
        ids16 = ids_v[buf, pl.ds(pos, L)]
        prev16 = ids_v[buf, pl.ds(pos - 1, L)]
        chg_b = ids16 != prev16
        cum_chg = plsc.cumsum(jnp.where(chg_b, 1, 0))
        rank16 = n_vec + cum_chg
        for g in range(N_CORES):
            for r in range(8):
                val = xv[buf, g, t_local, r, pl.ds(c_i * L, L)]
                plsc.addupdate_scatter(stg_v, [rank16, ucols[8 * g + r]], val)
        plsc.store_scatter(idt_v, [rank16], ids16)
        return n_vec + _gather16(cum_chg, fifteen)

    def _chunk_fast(buf, t_local, c_i, id0_vec, car):
        pos = IDS_OFF + t_local * 128 + c_i * L
        ids16 = ids_v[buf, pl.ds(pos, L)]
        prev16 = ids_v[buf, pl.ds(pos - 1, L)]
        uniform = jnp.all(ids16 == prev16)

        def _fast(car2):
            open_row = car2[0]
            accs = list(car2[1:])
            for g in range(N_CORES):
                for r in range(8):
                    u = 8 * g + r
                    accs[u] = accs[u] + xv[buf, g, t_local, r, pl.ds(c_i * L, L)]
            return (open_row, *accs)

        def _slow(car2):
            open_row = car2[0]
            accs = car2[1:]
            last_vec = _gather16(ids16, fifteen)
            trail_b = ids16 == last_vec
            keep_b = jnp.logical_not(trail_b)
            row16 = ids16 - id0_vec
            new_accs = []
            for g in range(N_CORES):
                for r in range(8):
                    u = 8 * g + r
                    val = xv[buf, g, t_local, r, pl.ds(c_i * L, L)]
                    plsc.addupdate_scatter(stg_v, [open_row, ucols[u]], accs[u])
                    plsc.addupdate_scatter(
                        stg_v, [row16, ucols[u]], val, mask=keep_b
                    )
                    new_accs.append(jnp.where(trail_b, val, fzero))
            return (last_vec - id0_vec, *new_accs)

        return lax.cond(uniform, _fast, _slow, car)

    def _issue(blk_i, buf):
        b = w + NW * blk_i
        pltpu.async_copy(
            ids_hbm.at[pl.ds(b * BLK_ATOMS, BLK_ATOMS)],
            ids_v.at[buf, pl.ds(IDS_OFF, BLK_ATOMS)],
            sem.at[buf],
        )
        pltpu.async_copy(x_hbm.at[:, pl.ds(b * TB, TB)], xv.at[buf], sem.at[buf])

    def _wait(buf):
        pltpu.make_async_copy(
            ids_hbm.at[pl.ds(0, BLK_ATOMS)],
            ids_v.at[buf, pl.ds(IDS_OFF, BLK_ATOMS)],
            sem.at[buf],
        ).wait()
        pltpu.make_async_copy(
            x_hbm.at[:, pl.ds(0, TB)], xv.at[buf], sem.at[buf]
        ).wait()

    nblk = jnp.where(w < EXTRA_W, FULL_BLK + 1, FULL_BLK)

    def _block(i, carry):
        buf = jnp.bitwise_and(i, 1)
        _wait(buf)

        @pl.when(i + 1 < nblk)
        def _prefetch():
            _issue(i + 1, 1 - buf)

        first16 = ids_v[buf, pl.ds(IDS_OFF, L)]
        id0_vec = _gather16(first16, zero16)
        last16 = ids_v[buf, pl.ds(IDS_OFF + BLK_ATOMS - L, L)]
        idlast_vec = _gather16(last16, fifteen)
        blk_range = jnp.max(idlast_vec - id0_vec) + 1

        def _idt(j, carry2):
            idt_v[pl.ds(j * L, L)] = jnp.minimum(
                id0_vec + (iota + j * L), dummy_vec
            )
            return carry2

        lax.fori_loop(0, STG // L, _idt, 0)

        def _normal(_):
            def _tile(t_local, car):
                for c_i in range(CHUNKS_PER_TILE):
                    car = _chunk_fast(buf, t_local, c_i, id0_vec, car)
                return car

            init = (jnp.full((L,), STG - 1, jnp.int32),) + tuple(
                fzero for _ in range(OUT_U)
            )
            car = lax.fori_loop(0, TB, _tile, init)
            open_row = car[0]
            accs = car[1:]
            for u in range(OUT_U):
                plsc.addupdate_scatter(stg_v, [open_row, ucols[u]], accs[u])
            return blk_range

        def _fallback(_):
            def _tile(t_local, n_vec):
                for c_i in range(CHUNKS_PER_TILE):
                    n_vec = _chunk_rank(buf, t_local, c_i, n_vec)
                return n_vec

            n_vec = lax.fori_loop(0, TB, _tile, zero16)
            return jnp.max(n_vec) + 1

        n_rows = lax.cond(blk_range > STG - 8, _fallback, _normal, 0)
        n_fc = (n_rows + 127) // 128

        def _flush(j, carry2):
            pltpu.sync_copy(
                stg_v.at[pl.ds(j * 128, 128)],
                acc_sh.at[idt_v.at[pl.ds(j * 128, 128)]],
                add=True,
            )
            return carry2

        lax.fori_loop(0, n_fc, _flush, 0)

        def _rezero(rr, carry2):
            stg_v[rr, :] = zrow
            return carry2

        lax.fori_loop(0, n_fc * 128, _rezero, 0)
        return carry

    _issue(0, 0)
    lax.fori_loop(0, nblk, _block, 0)
    plsc.subcore_barrier()

    pltpu.sync_copy(
        acc_sh.at[pl.ds(s * SEG_PER_TILE, SEG_PER_TILE)],
        out_hbm.at[c, pl.ds(s * SEG_PER_TILE, SEG_PER_TILE)],
    )


CMB_ROWS = 320
CMB_TAIL = N_SEG - (NW - 1) * CMB_ROWS


@functools.partial(
    pl.kernel,
    out_type=jax.ShapeDtypeStruct((N_SEG, OUT_U), jnp.float32),
    mesh=_mesh,
    scratch_types=[
        pltpu.VMEM((CMB_ROWS, OUT_U), jnp.float32),
        pltpu.VMEM((CMB_ROWS, OUT_U), jnp.float32),
    ],
    compiler_params=pltpu.CompilerParams(use_tc_tiling_on_sc=False, needs_layout_passes=False),
)
def _sc_combine(p_hbm, out_hbm, a_v, b_v):
    c = lax.axis_index("c")
    s = lax.axis_index("s")
    w = c * N_SUBCORES + s
    base = w * CMB_ROWS

    def _do(nrows):
        pltpu.sync_copy(p_hbm.at[0, pl.ds(base, nrows)], a_v.at[pl.ds(0, nrows)])
        pltpu.sync_copy(p_hbm.at[1, pl.ds(base, nrows)], b_v.at[pl.ds(0, nrows)])

        def _add(i, carry):
            a_v[i, :] = a_v[i, :] + b_v[i, :]
            return carry

        lax.fori_loop(0, nrows, _add, 0)
        pltpu.sync_copy(a_v.at[pl.ds(0, nrows)], out_hbm.at[pl.ds(base, nrows)])

    @pl.when(w < NW - 1)
    def _full():
        _do(CMB_ROWS)

    @pl.when(w == NW - 1)
    def _tail():
        _do(CMB_TAIL)


def kernel(ind_1, output):
    batch = ind_1[:, 0] if ind_1.ndim == 2 else ind_1
    ids = batch.astype(jnp.int32)
    x = output.reshape(T_TILES, 128, N_CORES, 8).transpose(2, 0, 3, 1)
    partials = _sc_segsum(ids, x)
    return _sc_combine(partials)


# speedup vs baseline: 4.1572x; 1.0313x over previous
import functools

import jax
import jax.numpy as jnp
from jax import lax
from jax.experimental import pallas as pl
from jax.experimental.pallas import tpu as pltpu
from jax.experimental.pallas import tpu_sc as plsc

N_ATOMS = 1600000
N_SEG = 10000
OUT_U = 16
L = 16

N_CORES = 2
N_SUBCORES = 16
NW = N_CORES * N_SUBCORES
T_TILES = N_ATOMS // 128
TB = 10
BLK_ATOMS = TB * 128
NBLOCKS = T_TILES // TB
FULL_BLK = NBLOCKS // NW
EXTRA_W = NBLOCKS - FULL_BLK * NW

CHUNKS_PER_TILE = 128 // L
STG = 1408
DUMMY = 10100
SEG_PER_TILE = 632
N_SEG_PAD = SEG_PER_TILE * N_SUBCORES

IDS_OFF = 8
IDS_LEN = IDS_OFF + BLK_ATOMS + 24

_mesh = plsc.VectorSubcoreMesh(core_axis_name="c", subcore_axis_name="s")

_GDN = lax.GatherDimensionNumbers(
    offset_dims=(), collapsed_slice_dims=(0,), start_index_map=(0,)
)


def _gather16(v, idx):
    return lax.gather(
        v, idx[:, None], _GDN, (1,),
        mode=lax.GatherScatterMode.PROMISE_IN_BOUNDS,
    )


@functools.partial(
    pl.kernel,
    out_type=jax.ShapeDtypeStruct((N_CORES, N_SEG_PAD, OUT_U), jnp.float32),
    mesh=_mesh,
    scratch_types=[
        pltpu.VMEM_SHARED((N_SEG_PAD, OUT_U), jnp.float32),
        pltpu.VMEM((2, N_CORES, TB, 8, 128), jnp.float32),
        pltpu.VMEM((2, IDS_LEN), jnp.int32),
        pltpu.VMEM((STG, OUT_U), jnp.float32),
        pltpu.VMEM((STG,), jnp.int32),
        pltpu.SemaphoreType.DMA((2,)),
    ],
    compiler_params=pltpu.CompilerParams(use_tc_tiling_on_sc=False, needs_layout_passes=False),
)
def _sc_segsum(ids_hbm, x_hbm, out_hbm, acc_sh, xv, ids_v, stg_v, idt_v, sem):
    c = lax.axis_index("c")
    s = lax.axis_index("s")
    w = c * N_SUBCORES + s

    iota = lax.iota(jnp.int32, L)
    zrow = jnp.zeros((OUT_U,), jnp.float32)
    dummy_vec = jnp.full((L,), DUMMY, jnp.int32)
    fifteen = jnp.full((L,), 15, jnp.int32)
    zero16 = jnp.zeros((L,), jnp.int32)
    fzero = jnp.zeros((L,), jnp.float32)
    ucols = [jnp.full((L,), u, jnp.int32) for u in range(OUT_U)]

    ids_v[0, pl.ds(0, L)] = jnp.full((L,), -1, jnp.int32)
    ids_v[1, pl.ds(0, L)] = jnp.full((L,), -1, jnp.int32)

    def _zero_stg(i, carry):
        stg_v[i, :] = zrow
        return carry

    lax.fori_loop(0, STG, _zero_stg, 0)

    def _dummy_idt(j, carry):
        idt_v[pl.ds(j * L, L)] = dummy_vec
        return carry

    lax.fori_loop(0, STG // L, _dummy_idt, 0)

    pltpu.sync_copy(stg_v.at[pl.ds(0, SEG_PER_TILE)],
                    acc_sh.at[pl.ds(s * SEG_PER_TILE, SEG_PER_TILE)])
    plsc.subcore_barrier()

    def _chunk_rank(buf, t_local, c_i, n_vec):
        pos = IDS_OFF + t_local * 128 + c_i * L
        ids16 = ids_v[buf, pl.ds(pos, L)]
        prev16 = ids_v[buf, pl.ds(pos - 1, L)]
        chg_b = ids16 != prev16
        cum_chg = plsc.cumsum(jnp.where(chg_b, 1, 0))
        rank16 = n_vec + cum_chg
        for g in range(N_CORES):
            for r in range(8):
                val = xv[buf, g, t_local, r, pl.ds(c_i * L, L)]
                plsc.addupdate_scatter(stg_v, [rank16, ucols[8 * g + r]], val)
        plsc.store_scatter(idt_v, [rank16], ids16)
        return n_vec + _gather16(cum_chg, fifteen)

    def _chunk_fast(buf, t_local, c_i, id0_vec, car):
        pos = IDS_OFF + t_local * 128 + c_i * L
        ids16 = ids_v[buf, pl.ds(pos, L)]
        prev16 = ids_v[buf, pl.ds(pos - 1, L)]
        uniform = jnp.all(ids16 == prev16)

        def _fast(car2):
            open_row = car2[0]
            accs = list(car2[1:])
            for g in range(N_CORES):
                for r in range(8):
                    u = 8 * g + r
                    accs[u] = accs[u] + xv[buf, g, t_local, r, pl.ds(c_i * L, L)]
            return (open_row, *accs)

        def _slow(car2):
            open_row = car2[0]
            accs = car2[1:]
            last_vec = _gather16(ids16, fifteen)
            trail_b = ids16 == last_vec
            keep_b = jnp.logical_not(trail_b)
            row16 = ids16 - id0_vec
            new_accs = []
            for g in range(N_CORES):
                for r in range(8):
                    u = 8 * g + r
                    val = xv[buf, g, t_local, r, pl.ds(c_i * L, L)]
                    plsc.addupdate_scatter(stg_v, [open_row, ucols[u]], accs[u])
                    plsc.addupdate_scatter(
                        stg_v, [row16, ucols[u]], val, mask=keep_b
                    )
                    new_accs.append(jnp.where(trail_b, val, fzero))
            return (last_vec - id0_vec, *new_accs)

        return lax.cond(uniform, _fast, _slow, car)

    def _issue(blk_i, buf):
        b = w + NW * blk_i
        pltpu.async_copy(
            ids_hbm.at[pl.ds(b * BLK_ATOMS, BLK_ATOMS)],
            ids_v.at[buf, pl.ds(IDS_OFF, BLK_ATOMS)],
            sem.at[buf],
        )
        pltpu.async_copy(x_hbm.at[:, pl.ds(b * TB, TB)], xv.at[buf], sem.at[buf])

    def _wait(buf):
        pltpu.make_async_copy(
            ids_hbm.at[pl.ds(0, BLK_ATOMS)],
            ids_v.at[buf, pl.ds(IDS_OFF, BLK_ATOMS)],
            sem.at[buf],
        ).wait()
        pltpu.make_async_copy(
            x_hbm.at[:, pl.ds(0, TB)], xv.at[buf], sem.at[buf]
        ).wait()

    nblk = jnp.where(w < EXTRA_W, FULL_BLK + 1, FULL_BLK)

    def _block(i, carry):
        buf = jnp.bitwise_and(i, 1)
        _wait(buf)

        @pl.when(i + 1 < nblk)
        def _prefetch():
            _issue(i + 1, 1 - buf)

        first16 = ids_v[buf, pl.ds(IDS_OFF, L)]
        id0_vec = _gather16(first16, zero16)
        last16 = ids_v[buf, pl.ds(IDS_OFF + BLK_ATOMS - L, L)]
        idlast_vec = _gather16(last16, fifteen)
        blk_range = jnp.max(idlast_vec - id0_vec) + 1

        def _idt(j, carry2):
            idt_v[pl.ds(j * L, L)] = jnp.minimum(
                id0_vec + (iota + j * L), dummy_vec
            )
            return carry2

        idt_hi = ((blk_range + 129) // 128) * (128 // L)
        lax.fori_loop(0, jnp.minimum(idt_hi, STG // L), _idt, 0)

        def _normal(_):
            def _tile(t_local, car):
                for c_i in range(CHUNKS_PER_TILE):
                    car = _chunk_fast(buf, t_local, c_i, id0_vec, car)
                return car

            init = (jnp.full((L,), STG - 1, jnp.int32),) + tuple(
                fzero for _ in range(OUT_U)
            )
            car = lax.fori_loop(0, TB, _tile, init)
            open_row = car[0]
            accs = car[1:]
            for u in range(OUT_U):
                plsc.addupdate_scatter(stg_v, [open_row, ucols[u]], accs[u])
            return blk_range

        def _fallback(_):
            def _tile(t_local, n_vec):
                for c_i in range(CHUNKS_PER_TILE):
                    n_vec = _chunk_rank(buf, t_local, c_i, n_vec)
                return n_vec

            n_vec = lax.fori_loop(0, TB, _tile, zero16)
            return jnp.max(n_vec) + 1

        n_rows = lax.cond(blk_range > STG - 8, _fallback, _normal, 0)
        n_fc = (n_rows + 127) // 128

        def _flush(j, carry2):
            pltpu.sync_copy(
                stg_v.at[pl.ds(j * 128, 128)],
                acc_sh.at[idt_v.at[pl.ds(j * 128, 128)]],
                add=True,
            )
            return carry2

        lax.fori_loop(0, n_fc, _flush, 0)

        def _rezero(rr, carry2):
            stg_v[rr, :] = zrow
            return carry2

        lax.fori_loop(0, n_fc * 128, _rezero, 0)
        return carry

    _issue(0, 0)
    lax.fori_loop(0, nblk, _block, 0)
    plsc.subcore_barrier()

    pltpu.sync_copy(
        acc_sh.at[pl.ds(s * SEG_PER_TILE, SEG_PER_TILE)],
        out_hbm.at[c, pl.ds(s * SEG_PER_TILE, SEG_PER_TILE)],
    )


CMB_ROWS = 320
CMB_TAIL = N_SEG - (NW - 1) * CMB_ROWS


@functools.partial(
    pl.kernel,
    out_type=jax.ShapeDtypeStruct((N_SEG, OUT_U), jnp.float32),
    mesh=_mesh,
    scratch_types=[
        pltpu.VMEM((CMB_ROWS, OUT_U), jnp.float32),
        pltpu.VMEM((CMB_ROWS, OUT_U), jnp.float32),
    ],
    compiler_params=pltpu.CompilerParams(use_tc_tiling_on_sc=False, needs_layout_passes=False),
)
def _sc_combine(p_hbm, out_hbm, a_v, b_v):
    c = lax.axis_index("c")
    s = lax.axis_index("s")
    w = c * N_SUBCORES + s
    base = w * CMB_ROWS

    def _do(nrows):
        pltpu.sync_copy(p_hbm.at[0, pl.ds(base, nrows)], a_v.at[pl.ds(0, nrows)])
        pltpu.sync_copy(p_hbm.at[1, pl.ds(base, nrows)], b_v.at[pl.ds(0, nrows)])

        def _add(i, carry):
            a_v[i, :] = a_v[i, :] + b_v[i, :]
            return carry

        lax.fori_loop(0, nrows, _add, 0)
        pltpu.sync_copy(a_v.at[pl.ds(0, nrows)], out_hbm.at[pl.ds(base, nrows)])

    @pl.when(w < NW - 1)
    def _full():
        _do(CMB_ROWS)

    @pl.when(w == NW - 1)
    def _tail():
        _do(CMB_TAIL)


def kernel(ind_1, output):
    batch = ind_1[:, 0] if ind_1.ndim == 2 else ind_1
    ids = batch.astype(jnp.int32)
    x = output.reshape(T_TILES, 128, N_CORES, 8).transpose(2, 0, 3, 1)
    partials = _sc_segsum(ids, x)
    return _sc_combine(partials)


# speedup vs baseline: 4.2451x; 1.0212x over previous
import functools

import jax
import jax.numpy as jnp
from jax import lax
from jax.experimental import pallas as pl
from jax.experimental.pallas import tpu as pltpu
from jax.experimental.pallas import tpu_sc as plsc

N_ATOMS = 1600000
N_SEG = 10000
OUT_U = 16
L = 16

N_CORES = 2
N_SUBCORES = 16
NW = N_CORES * N_SUBCORES
T_TILES = N_ATOMS // 128
TB = 10
BLK_ATOMS = TB * 128
NBLOCKS = T_TILES // TB
FULL_BLK = NBLOCKS // NW
EXTRA_W = NBLOCKS - FULL_BLK * NW

CHUNKS_PER_TILE = 128 // L
STG = 1408
DUMMY = 10100
SEG_PER_TILE = 632
N_SEG_PAD = SEG_PER_TILE * N_SUBCORES

IDS_OFF = 8
IDS_LEN = IDS_OFF + BLK_ATOMS + 24

_mesh = plsc.VectorSubcoreMesh(core_axis_name="c", subcore_axis_name="s")

_GDN = lax.GatherDimensionNumbers(
    offset_dims=(), collapsed_slice_dims=(0,), start_index_map=(0,)
)


def _gather16(v, idx):
    return lax.gather(
        v, idx[:, None], _GDN, (1,),
        mode=lax.GatherScatterMode.PROMISE_IN_BOUNDS,
    )


@functools.partial(
    pl.kernel,
    out_type=jax.ShapeDtypeStruct((N_CORES, N_SEG_PAD, OUT_U), jnp.float32),
    mesh=_mesh,
    scratch_types=[
        pltpu.VMEM_SHARED((N_SEG_PAD, OUT_U), jnp.float32),
        pltpu.VMEM((2, N_CORES, TB, 8, 128), jnp.float32),
        pltpu.VMEM((2, IDS_LEN), jnp.int32),
        pltpu.VMEM((2, STG, OUT_U), jnp.float32),
        pltpu.VMEM((2, STG), jnp.int32),
        pltpu.SemaphoreType.DMA((2,)),
        pltpu.SemaphoreType.DMA((2,)),
    ],
    compiler_params=pltpu.CompilerParams(use_tc_tiling_on_sc=False, needs_layout_passes=False),
)
def _sc_segsum(ids_hbm, x_hbm, out_hbm, acc_sh, xv, ids_v, stg2_v, idt2_v,
               sem, fsem):
    c = lax.axis_index("c")
    s = lax.axis_index("s")
    w = c * N_SUBCORES + s

    iota = lax.iota(jnp.int32, L)
    zrow = jnp.zeros((OUT_U,), jnp.float32)
    dummy_vec = jnp.full((L,), DUMMY, jnp.int32)
    fifteen = jnp.full((L,), 15, jnp.int32)
    zero16 = jnp.zeros((L,), jnp.int32)
    fzero = jnp.zeros((L,), jnp.float32)
    ucols = [jnp.full((L,), u, jnp.int32) for u in range(OUT_U)]

    ids_v[0, pl.ds(0, L)] = jnp.full((L,), -1, jnp.int32)
    ids_v[1, pl.ds(0, L)] = jnp.full((L,), -1, jnp.int32)

    def _zero_stg(i, carry):
        stg2_v[0, i, :] = zrow
        stg2_v[1, i, :] = zrow
        return carry

    lax.fori_loop(0, STG, _zero_stg, 0)

    pltpu.sync_copy(stg2_v.at[0, pl.ds(0, SEG_PER_TILE)],
                    acc_sh.at[pl.ds(s * SEG_PER_TILE, SEG_PER_TILE)])
    plsc.subcore_barrier()

    def _chunk_rank(buf, stg, idt, t_local, c_i, n_vec):
        pos = IDS_OFF + t_local * 128 + c_i * L
        ids16 = ids_v[buf, pl.ds(pos, L)]
        prev16 = ids_v[buf, pl.ds(pos - 1, L)]
        chg_b = ids16 != prev16
        cum_chg = plsc.cumsum(jnp.where(chg_b, 1, 0))
        rank16 = n_vec + cum_chg
        for g in range(N_CORES):
            for r in range(8):
                val = xv[buf, g, t_local, r, pl.ds(c_i * L, L)]
                plsc.addupdate_scatter(stg, [rank16, ucols[8 * g + r]], val)
        plsc.store_scatter(idt, [rank16], ids16)
        return n_vec + _gather16(cum_chg, fifteen)

    def _chunk_fast(buf, stg, t_local, c_i, id0_vec, car):
        pos = IDS_OFF + t_local * 128 + c_i * L
        ids16 = ids_v[buf, pl.ds(pos, L)]
        prev16 = ids_v[buf, pl.ds(pos - 1, L)]
        uniform = jnp.all(ids16 == prev16)

        def _fast(car2):
            open_row = car2[0]
            accs = list(car2[1:])
            for g in range(N_CORES):
                for r in range(8):
                    u = 8 * g + r
                    accs[u] = accs[u] + xv[buf, g, t_local, r, pl.ds(c_i * L, L)]
            return (open_row, *accs)

        def _slow(car2):
            open_row = car2[0]
            accs = car2[1:]
            last_vec = _gather16(ids16, fifteen)
            trail_b = ids16 == last_vec
            keep_b = jnp.logical_not(trail_b)
            row16 = ids16 - id0_vec
            new_accs = []
            for g in range(N_CORES):
                for r in range(8):
                    u = 8 * g + r
                    val = xv[buf, g, t_local, r, pl.ds(c_i * L, L)]
                    plsc.addupdate_scatter(stg, [open_row, ucols[u]], accs[u])
                    plsc.addupdate_scatter(
                        stg, [row16, ucols[u]], val, mask=keep_b
                    )
                    new_accs.append(jnp.where(trail_b, val, fzero))
            return (last_vec - id0_vec, *new_accs)

        return lax.cond(uniform, _fast, _slow, car)

    def _issue(blk_i, buf):
        b = w + NW * blk_i
        pltpu.async_copy(
            ids_hbm.at[pl.ds(b * BLK_ATOMS, BLK_ATOMS)],
            ids_v.at[buf, pl.ds(IDS_OFF, BLK_ATOMS)],
            sem.at[buf],
        )
        pltpu.async_copy(x_hbm.at[:, pl.ds(b * TB, TB)], xv.at[buf], sem.at[buf])

    def _wait(buf):
        pltpu.make_async_copy(
            ids_hbm.at[pl.ds(0, BLK_ATOMS)],
            ids_v.at[buf, pl.ds(IDS_OFF, BLK_ATOMS)],
            sem.at[buf],
        ).wait()
        pltpu.make_async_copy(
            x_hbm.at[:, pl.ds(0, TB)], xv.at[buf], sem.at[buf]
        ).wait()

    nblk = jnp.where(w < EXTRA_W, FULL_BLK + 1, FULL_BLK)

    def _drain_flush(buf, pn):
        def _d(j, carry2):
            pltpu.make_async_copy(
                stg2_v.at[0, pl.ds(0, 128)],
                acc_sh.at[pl.ds(0, 128)],
                fsem.at[buf],
            ).wait()
            return carry2

        lax.fori_loop(0, pn, _d, 0)

    def _block(i, carry):
        pn0, pn1 = carry
        buf = jnp.bitwise_and(i, 1)
        stg = stg2_v.at[buf]
        idt = idt2_v.at[buf]
        pn = jnp.where(buf == 0, pn0, pn1)
        _drain_flush(buf, pn)

        def _rezero(rr, carry2):
            stg2_v[buf, rr, :] = zrow
            return carry2

        lax.fori_loop(0, pn * 128, _rezero, 0)
        _wait(buf)

        @pl.when(i + 1 < nblk)
        def _prefetch():
            _issue(i + 1, 1 - buf)

        first16 = ids_v[buf, pl.ds(IDS_OFF, L)]
        id0_vec = _gather16(first16, zero16)
        last16 = ids_v[buf, pl.ds(IDS_OFF + BLK_ATOMS - L, L)]
        idlast_vec = _gather16(last16, fifteen)
        blk_range = jnp.max(idlast_vec - id0_vec) + 1

        def _idt(j, carry2):
            idt2_v[buf, pl.ds(j * L, L)] = jnp.minimum(
                id0_vec + (iota + j * L), dummy_vec
            )
            return carry2

        idt_hi = ((blk_range + 129) // 128) * (128 // L)
        lax.fori_loop(0, jnp.minimum(idt_hi, STG // L), _idt, 0)

        def _normal(_):
            def _tile(t_local, car):
                for c_i in range(CHUNKS_PER_TILE):
                    car = _chunk_fast(buf, stg, t_local, c_i, id0_vec, car)
                return car

            init = (jnp.full((L,), STG - 1, jnp.int32),) + tuple(
                fzero for _ in range(OUT_U)
            )
            car = lax.fori_loop(0, TB, _tile, init)
            open_row = car[0]
            accs = car[1:]
            for u in range(OUT_U):
                plsc.addupdate_scatter(stg, [open_row, ucols[u]], accs[u])
            return blk_range

        def _fallback(_):
            def _tile(t_local, n_vec):
                for c_i in range(CHUNKS_PER_TILE):
                    n_vec = _chunk_rank(buf, stg, idt, t_local, c_i, n_vec)
                return n_vec

            n_vec = lax.fori_loop(0, TB, _tile, zero16)
            return jnp.max(n_vec) + 1

        n_rows = lax.cond(blk_range > STG - 8, _fallback, _normal, 0)
        n_fc = (n_rows + 127) // 128

        def _flush(j, carry2):
            pltpu.async_copy(
                stg2_v.at[buf, pl.ds(j * 128, 128)],
                acc_sh.at[idt2_v.at[buf, pl.ds(j * 128, 128)]],
                fsem.at[buf],
                add=True,
            )
            return carry2

        lax.fori_loop(0, n_fc, _flush, 0)
        return (jnp.where(buf == 0, n_fc, pn0), jnp.where(buf == 0, pn1, n_fc))

    _issue(0, 0)
    pn0, pn1 = lax.fori_loop(0, nblk, _block, (jnp.int32(0), jnp.int32(0)))
    _drain_flush(0, pn0)
    _drain_flush(1, pn1)
    plsc.subcore_barrier()

    pltpu.sync_copy(
        acc_sh.at[pl.ds(s * SEG_PER_TILE, SEG_PER_TILE)],
        out_hbm.at[c, pl.ds(s * SEG_PER_TILE, SEG_PER_TILE)],
    )


CMB_ROWS = 320
CMB_TAIL = N_SEG - (NW - 1) * CMB_ROWS


@functools.partial(
    pl.kernel,
    out_type=jax.ShapeDtypeStruct((N_SEG, OUT_U), jnp.float32),
    mesh=_mesh,
    scratch_types=[
        pltpu.VMEM((CMB_ROWS, OUT_U), jnp.float32),
        pltpu.VMEM((CMB_ROWS, OUT_U), jnp.float32),
    ],
    compiler_params=pltpu.CompilerParams(use_tc_tiling_on_sc=False, needs_layout_passes=False),
)
def _sc_combine(p_hbm, out_hbm, a_v, b_v):
    c = lax.axis_index("c")
    s = lax.axis_index("s")
    w = c * N_SUBCORES + s
    base = w * CMB_ROWS

    def _do(nrows):
        pltpu.sync_copy(p_hbm.at[0, pl.ds(base, nrows)], a_v.at[pl.ds(0, nrows)])
        pltpu.sync_copy(p_hbm.at[1, pl.ds(base, nrows)], b_v.at[pl.ds(0, nrows)])

        def _add(i, carry):
            a_v[i, :] = a_v[i, :] + b_v[i, :]
            return carry

        lax.fori_loop(0, nrows, _add, 0)
        pltpu.sync_copy(a_v.at[pl.ds(0, nrows)], out_hbm.at[pl.ds(base, nrows)])

    @pl.when(w < NW - 1)
    def _full():
        _do(CMB_ROWS)

    @pl.when(w == NW - 1)
    def _tail():
        _do(CMB_TAIL)


def kernel(ind_1, output):
    batch = ind_1[:, 0] if ind_1.ndim == 2 else ind_1
    ids = batch.astype(jnp.int32)
    x = output.reshape(T_TILES, 128, N_CORES, 8).transpose(2, 0, 3, 1)
    partials = _sc_segsum(ids, x)
    return _sc_combine(partials)
